# baseline (speedup 1.0000x reference)
import functools

import jax
import jax.numpy as jnp
from jax import lax
from jax.experimental import pallas as pl
from jax.experimental.pallas import tpu as pltpu
from jax.experimental.pallas import tpu_sc as plsc

_B = 16384
_D = 32
_H = 64
_NC = 2
_NS = 16
_NW = _NC * _NS
_BPW = _B // _NW
_CH = 256
_NCH = _BPW // _CH
_PK = 128 // _D
_NROW = 1000000 // _PK


@functools.cache
def _sc_gather_fn():
    mesh = plsc.VectorSubcoreMesh(
        core_axis_name="c", subcore_axis_name="s",
        num_cores=_NC, num_subcores=_NS,
    )

    @functools.partial(
        pl.kernel,
        out_type=(
            jax.ShapeDtypeStruct((_B, 128), jnp.float32),
            jax.ShapeDtypeStruct((_B, 128), jnp.float32),
        ),
        mesh=mesh,
        scratch_types=[
            pltpu.VMEM((_BPW,), jnp.int32),
            pltpu.VMEM((_BPW,), jnp.int32),
            pltpu.VMEM((_CH,), jnp.int32),
            pltpu.VMEM((_CH,), jnp.int32),
            pltpu.VMEM((_CH, 128), jnp.float32),
            pltpu.VMEM((_CH, 128), jnp.float32),
            pltpu.SemaphoreType.DMA,
            pltpu.SemaphoreType.DMA,
        ],
    )
    def sc_gather(U_hbm, V_hbm, u_hbm, i_hbm, eu_hbm, ev_hbm,
                  uidx, iidx, urow, irow, gu, gv, sem_u, sem_v):
        wid = lax.axis_index("s") * _NC + lax.axis_index("c")
        base = wid * _BPW
        pltpu.sync_copy(u_hbm.at[pl.ds(base, _BPW)], uidx)
        pltpu.sync_copy(i_hbm.at[pl.ds(base, _BPW)], iidx)
        for c in range(_NCH):
            for k in range(_CH // 16):
                s = pl.ds(k * 16, 16)
                urow[s] = jax.lax.shift_right_logical(
                    uidx[pl.ds(c * _CH + k * 16, 16)], 2)
                irow[s] = jax.lax.shift_right_logical(
                    iidx[pl.ds(c * _CH + k * 16, 16)], 2)
            cu = pltpu.async_copy(U_hbm.at[urow], gu, sem_u)
            cv = pltpu.async_copy(V_hbm.at[irow], gv, sem_v)
            cu.wait()
            pltpu.sync_copy(gu, eu_hbm.at[pl.ds(base + c * _CH, _CH)])
            cv.wait()
            pltpu.sync_copy(gv, ev_hbm.at[pl.ds(base + c * _CH, _CH)])

    return sc_gather


_BLK = 2048


def _mlp_body(u_ref, i_ref, eu_ref, ev_ref,
              w1a_ref, w1b_ref, b1_ref, w2_ref, b2_ref, o_ref):
    ku = jnp.transpose(jnp.reshape(u_ref[...], (1, _BLK)) & (_PK - 1))
    ki = jnp.transpose(jnp.reshape(i_ref[...], (1, _BLK)) & (_PK - 1))
    e = eu_ref[...]
    f = ev_ref[...]
    eu = jnp.zeros((_BLK, _D), jnp.float32)
    ev = jnp.zeros((_BLK, _D), jnp.float32)
    for k in range(_PK):
        sl = slice(_D * k, _D * (k + 1))
        eu = eu + jnp.where(ku == k, e[:, sl], 0.0)
        ev = ev + jnp.where(ki == k, f[:, sl], 0.0)
    h = jnp.dot(eu, w1a_ref[...], preferred_element_type=jnp.float32)
    h = h + jnp.dot(ev, w1b_ref[...], preferred_element_type=jnp.float32)
    h = jnp.maximum(h + b1_ref[...], 0.0)
    z = jnp.sum(h * w2_ref[...], axis=1) + b2_ref[0, 0]
    o_ref[...] = 1.0 / (1.0 + jnp.exp(-z))


_mlp = pl.pallas_call(
    _mlp_body,
    grid=(_B // _BLK,),
    in_specs=[
        pl.BlockSpec((1, 1, _BLK), lambda j: (j, 0, 0)),
        pl.BlockSpec((1, 1, _BLK), lambda j: (j, 0, 0)),
        pl.BlockSpec((_BLK, 128), lambda j: (j, 0)),
        pl.BlockSpec((_BLK, 128), lambda j: (j, 0)),
        pl.BlockSpec((_D, _H), lambda j: (0, 0)),
        pl.BlockSpec((_D, _H), lambda j: (0, 0)),
        pl.BlockSpec((1, _H), lambda j: (0, 0)),
        pl.BlockSpec((1, _H), lambda j: (0, 0)),
        pl.BlockSpec((1, 1), lambda j: (0, 0)),
    ],
    out_specs=pl.BlockSpec((_BLK,), lambda j: (j,)),
    out_shape=jax.ShapeDtypeStruct((_B,), jnp.float32),
)


def kernel(u, i, U, V, W1, b1, W2, b2):
    u = u.astype(jnp.int32)
    i = i.astype(jnp.int32)
    U128 = U.reshape(_NROW, 128)
    V128 = V.reshape(_NROW, 128)
    eu128, ev128 = _sc_gather_fn()(U128, V128, u, i)
    w1a = W1[:, :_D].T
    w1b = W1[:, _D:].T
    return _mlp(u.reshape(_B // _BLK, 1, _BLK), i.reshape(_B // _BLK, 1, _BLK),
                eu128, ev128, w1a, w1b,
                b1.reshape(1, _H), W2, b2.reshape(1, 1))


# speedup vs baseline: 2.6440x; 2.6440x over previous
import functools

import jax
import jax.numpy as jnp
from jax import lax
from jax.experimental import pallas as pl
from jax.experimental.pallas import tpu as pltpu
from jax.experimental.pallas import tpu_sc as plsc

_NV = 1000000
_B = 16384
_D = 32
_H = 64
_NC = 2
_NS = 16
_NW = _NC * _NS
_BPW = _B // _NW
_CH = 256
_NCH = _BPW // _CH
_PK = 128 // _D
_PBLK = 2048
_NCOLB = (_NV + _PBLK - 1) // _PBLK
_PGRID = (_NCOLB + _PK - 1) // _PK
_NROW = _PBLK * _PGRID


def _pack_body(u0_ref, u1_ref, u2_ref, u3_ref, o_ref):
    eye = (jax.lax.broadcasted_iota(jnp.int32, (128, 128), 0)
           == jax.lax.broadcasted_iota(jnp.int32, (128, 128), 1)).astype(jnp.float32)
    ubig = jnp.concatenate(
        [u0_ref[...], u1_ref[...], u2_ref[...], u3_ref[...]], axis=0)
    o_ref[...] = jax.lax.dot_general(
        ubig, eye, (((0,), (0,)), ((), ())),
        preferred_element_type=jnp.float32)


_pack = pl.pallas_call(
    _pack_body,
    grid=(_PGRID,),
    in_specs=[pl.BlockSpec(
        (_D, _PBLK),
        lambda j, k=k: (0, jnp.minimum(_PK * j + k, _NCOLB - 1)))
        for k in range(_PK)],
    out_specs=pl.BlockSpec((_PBLK, 128), lambda j: (j, 0)),
    out_shape=jax.ShapeDtypeStruct((_NROW, 128), jnp.float32),
    compiler_params=pltpu.CompilerParams(fuse_transposed_lhs_in_matmul=True),
)


@functools.cache
def _sc_gather_fn():
    mesh = plsc.VectorSubcoreMesh(
        core_axis_name="c", subcore_axis_name="s",
        num_cores=_NC, num_subcores=_NS,
    )

    @functools.partial(
        pl.kernel,
        out_type=(
            jax.ShapeDtypeStruct((_B, 128), jnp.float32),
            jax.ShapeDtypeStruct((_B, 128), jnp.float32),
        ),
        mesh=mesh,
        compiler_params=pltpu.CompilerParams(use_tc_tiling_on_sc=True),
        scratch_types=[
            pltpu.VMEM((_BPW,), jnp.int32),
            pltpu.VMEM((_BPW,), jnp.int32),
            pltpu.VMEM((_CH,), jnp.int32),
            pltpu.VMEM((_CH,), jnp.int32),
            pltpu.VMEM((_CH, 128), jnp.float32),
            pltpu.VMEM((_CH, 128), jnp.float32),
            pltpu.SemaphoreType.DMA,
            pltpu.SemaphoreType.DMA,
        ],
    )
    def sc_gather(U_hbm, V_hbm, u_hbm, i_hbm, eu_hbm, ev_hbm,
                  uidx, iidx, urow, irow, gu, gv, sem_u, sem_v):
        wid = lax.axis_index("s") * _NC + lax.axis_index("c")
        base = wid * _BPW
        pltpu.sync_copy(u_hbm.at[pl.ds(base, _BPW)], uidx)
        pltpu.sync_copy(i_hbm.at[pl.ds(base, _BPW)], iidx)
        for c in range(_NCH):
            for k in range(_CH // 16):
                s = pl.ds(k * 16, 16)
                uu = uidx[pl.ds(c * _CH + k * 16, 16)]
                ii = iidx[pl.ds(c * _CH + k * 16, 16)]
                urow[s] = ((jax.lax.shift_right_logical(uu, 2) & ~2047)
                           | (uu & 2047))
                irow[s] = ((jax.lax.shift_right_logical(ii, 2) & ~2047)
                           | (ii & 2047))
            cu = pltpu.async_copy(U_hbm.at[urow], gu, sem_u)
            cv = pltpu.async_copy(V_hbm.at[irow], gv, sem_v)
            cu.wait()
            pltpu.sync_copy(gu, eu_hbm.at[pl.ds(base + c * _CH, _CH)])
            cv.wait()
            pltpu.sync_copy(gv, ev_hbm.at[pl.ds(base + c * _CH, _CH)])

    return sc_gather


_BLK = 2048


def _mlp_body(u_ref, i_ref, eu_ref, ev_ref,
              w1a_ref, w1b_ref, b1_ref, w2_ref, b2_ref, o_ref):
    ku = (jnp.transpose(jnp.reshape(u_ref[...], (1, _BLK))) >> 11) & 3
    ki = (jnp.transpose(jnp.reshape(i_ref[...], (1, _BLK))) >> 11) & 3
    e = eu_ref[...]
    f = ev_ref[...]
    eu = jnp.zeros((_BLK, _D), jnp.float32)
    ev = jnp.zeros((_BLK, _D), jnp.float32)
    for k in range(_PK):
        sl = slice(_D * k, _D * (k + 1))
        eu = eu + jnp.where(ku == k, e[:, sl], 0.0)
        ev = ev + jnp.where(ki == k, f[:, sl], 0.0)
    h = jnp.dot(eu, w1a_ref[...], preferred_element_type=jnp.float32)
    h = h + jnp.dot(ev, w1b_ref[...], preferred_element_type=jnp.float32)
    h = jnp.maximum(h + b1_ref[...], 0.0)
    z = jnp.sum(h * w2_ref[...], axis=1) + b2_ref[0, 0]
    o_ref[...] = 1.0 / (1.0 + jnp.exp(-z))


_mlp = pl.pallas_call(
    _mlp_body,
    grid=(_B // _BLK,),
    in_specs=[
        pl.BlockSpec((1, 1, _BLK), lambda j: (j, 0, 0)),
        pl.BlockSpec((1, 1, _BLK), lambda j: (j, 0, 0)),
        pl.BlockSpec((_BLK, 128), lambda j: (j, 0)),
        pl.BlockSpec((_BLK, 128), lambda j: (j, 0)),
        pl.BlockSpec((_D, _H), lambda j: (0, 0)),
        pl.BlockSpec((_D, _H), lambda j: (0, 0)),
        pl.BlockSpec((1, _H), lambda j: (0, 0)),
        pl.BlockSpec((1, _H), lambda j: (0, 0)),
        pl.BlockSpec((1, 1), lambda j: (0, 0)),
    ],
    out_specs=pl.BlockSpec((_BLK,), lambda j: (j,)),
    out_shape=jax.ShapeDtypeStruct((_B,), jnp.float32),
)


def kernel(u, i, U, V, W1, b1, W2, b2):
    u = u.astype(jnp.int32)
    i = i.astype(jnp.int32)
    UT, VT = U.T, V.T
    U128 = _pack(UT, UT, UT, UT)
    V128 = _pack(VT, VT, VT, VT)
    eu128, ev128 = _sc_gather_fn()(U128, V128, u, i)
    w1a = W1[:, :_D].T
    w1b = W1[:, _D:].T
    return _mlp(u.reshape(_B // _BLK, 1, _BLK), i.reshape(_B // _BLK, 1, _BLK),
                eu128, ev128, w1a, w1b,
                b1.reshape(1, _H), W2, b2.reshape(1, 1))


# speedup vs baseline: 2.8420x; 1.0749x over previous
import functools

import jax
import jax.numpy as jnp
from jax import lax
from jax.experimental import pallas as pl
from jax.experimental.pallas import tpu as pltpu
from jax.experimental.pallas import tpu_sc as plsc

_NV = 1000000
_B = 16384
_D = 32
_H = 64
_NC = 2
_NS = 16
_NW = _NC * _NS
_BPW = _B // _NW
_CH = 256
_NCH = _BPW // _CH
_PK = 128 // _D
_PBLK = 2048
_NCOLB = (_NV + _PBLK - 1) // _PBLK
_PGRID = (_NCOLB + _PK - 1) // _PK
_NROW = _PBLK * _PGRID


def _pack_body(u0_ref, u1_ref, u2_ref, u3_ref, o_ref):
    eye = (jax.lax.broadcasted_iota(jnp.int32, (128, 128), 0)
           == jax.lax.broadcasted_iota(jnp.int32, (128, 128), 1)).astype(jnp.float32)
    ubig = jnp.concatenate(
        [u0_ref[...], u1_ref[...], u2_ref[...], u3_ref[...]], axis=0)
    o_ref[...] = jax.lax.dot_general(
        ubig, eye, (((0,), (0,)), ((), ())),
        preferred_element_type=jnp.float32)


_pack = pl.pallas_call(
    _pack_body,
    grid=(_PGRID,),
    in_specs=[pl.BlockSpec(
        (_D, _PBLK),
        lambda j, k=k: (0, jnp.minimum(_PK * j + k, _NCOLB - 1)))
        for k in range(_PK)],
    out_specs=pl.BlockSpec((_PBLK, 128), lambda j: (j, 0)),
    out_shape=jax.ShapeDtypeStruct((_NROW, 128), jnp.float32),
    compiler_params=pltpu.CompilerParams(fuse_transposed_lhs_in_matmul=True),
)


@functools.cache
def _sc_gather_fn():
    mesh = plsc.VectorSubcoreMesh(
        core_axis_name="c", subcore_axis_name="s",
        num_cores=_NC, num_subcores=_NS,
    )

    @functools.partial(
        pl.kernel,
        out_type=(
            jax.ShapeDtypeStruct((_B, 128), jnp.float32),
            jax.ShapeDtypeStruct((_B, 128), jnp.float32),
        ),
        mesh=mesh,
        compiler_params=pltpu.CompilerParams(use_tc_tiling_on_sc=True),
        scratch_types=[
            pltpu.VMEM((_BPW,), jnp.int32),
            pltpu.VMEM((_BPW,), jnp.int32),
            pltpu.VMEM((_CH,), jnp.int32),
            pltpu.VMEM((_CH,), jnp.int32),
            pltpu.VMEM((_CH, 128), jnp.float32),
            pltpu.VMEM((_CH, 128), jnp.float32),
            pltpu.SemaphoreType.DMA,
            pltpu.SemaphoreType.DMA,
        ],
    )
    def sc_gather(U_hbm, V_hbm, u_hbm, i_hbm, eu_hbm, ev_hbm,
                  uidx, iidx, urow, irow, gu, gv, sem_u, sem_v):
        wid = lax.axis_index("s") * _NC + lax.axis_index("c")
        base = wid * _BPW
        pltpu.sync_copy(u_hbm.at[pl.ds(base, _BPW)], uidx)
        pltpu.sync_copy(i_hbm.at[pl.ds(base, _BPW)], iidx)
        for c in range(_NCH):
            for k in range(_CH // 16):
                s = pl.ds(k * 16, 16)
                uu = uidx[pl.ds(c * _CH + k * 16, 16)]
                ii = iidx[pl.ds(c * _CH + k * 16, 16)]
                urow[s] = ((jax.lax.shift_right_logical(uu, 2) & ~2047)
                           | (uu & 2047))
                irow[s] = ((jax.lax.shift_right_logical(ii, 2) & ~2047)
                           | (ii & 2047))
            cu = pltpu.async_copy(U_hbm.at[urow], gu, sem_u)
            cv = pltpu.async_copy(V_hbm.at[irow], gv, sem_v)
            cu.wait()
            pltpu.sync_copy(gu, eu_hbm.at[pl.ds(base + c * _CH, _CH)])
            cv.wait()
            pltpu.sync_copy(gv, ev_hbm.at[pl.ds(base + c * _CH, _CH)])

    return sc_gather


_BLK = 2048


def _mlp_body(u_ref, i_ref, eu_ref, ev_ref,
              w1a_ref, w1b_ref, b1_ref, w2_ref, b2_ref, o_ref):
    ku = (jnp.transpose(jnp.reshape(u_ref[...], (1, _BLK))) >> 11) & 3
    ki = (jnp.transpose(jnp.reshape(i_ref[...], (1, _BLK))) >> 11) & 3
    lane = jax.lax.broadcasted_iota(jnp.int32, (1, 128), 1) >> 5
    mu = (ku == lane).astype(jnp.float32)
    mi = (ki == lane).astype(jnp.float32)
    e = eu_ref[...] * mu
    f = ev_ref[...] * mi
    h = jnp.dot(e, w1a_ref[...], preferred_element_type=jnp.float32)
    h = h + jnp.dot(f, w1b_ref[...], preferred_element_type=jnp.float32)
    h = jnp.maximum(h + b1_ref[...], 0.0)
    z = jnp.sum(h * w2_ref[...], axis=1) + b2_ref[0, 0]
    o_ref[...] = 1.0 / (1.0 + jnp.exp(-z))


_mlp = pl.pallas_call(
    _mlp_body,
    grid=(_B // _BLK,),
    in_specs=[
        pl.BlockSpec((1, 1, _BLK), lambda j: (j, 0, 0)),
        pl.BlockSpec((1, 1, _BLK), lambda j: (j, 0, 0)),
        pl.BlockSpec((_BLK, 128), lambda j: (j, 0)),
        pl.BlockSpec((_BLK, 128), lambda j: (j, 0)),
        pl.BlockSpec((128, _H), lambda j: (0, 0)),
        pl.BlockSpec((128, _H), lambda j: (0, 0)),
        pl.BlockSpec((1, _H), lambda j: (0, 0)),
        pl.BlockSpec((1, _H), lambda j: (0, 0)),
        pl.BlockSpec((1, 1), lambda j: (0, 0)),
    ],
    out_specs=pl.BlockSpec((_BLK,), lambda j: (j,)),
    out_shape=jax.ShapeDtypeStruct((_B,), jnp.float32),
)


def kernel(u, i, U, V, W1, b1, W2, b2):
    u = u.astype(jnp.int32)
    i = i.astype(jnp.int32)
    UT, VT = U.T, V.T
    U128 = _pack(UT, UT, UT, UT)
    V128 = _pack(VT, VT, VT, VT)
    eu128, ev128 = _sc_gather_fn()(U128, V128, u, i)
    w1a = jnp.tile(W1[:, :_D].T, (_PK, 1))
    w1b = jnp.tile(W1[:, _D:].T, (_PK, 1))
    return _mlp(u.reshape(_B // _BLK, 1, _BLK), i.reshape(_B // _BLK, 1, _BLK),
                eu128, ev128, w1a, w1b,
                b1.reshape(1, _H), W2, b2.reshape(1, 1))


# speedup vs baseline: 4.2995x; 1.5129x over previous
import functools

import jax
import jax.numpy as jnp
from jax import lax
from jax.experimental import pallas as pl
from jax.experimental.pallas import tpu as pltpu
from jax.experimental.pallas import tpu_sc as plsc

_NV = 1000000
_B = 16384
_D = 32
_H = 64
_NC = 2
_NS = 16
_NW = _NC * _NS
_BPW = _B // _NW
_CH = 256
_NCH = _BPW // _CH
_PK = 128 // _D
_PBLK = 8192
_NCOLB = (_NV + _PBLK - 1) // _PBLK
_PGRID = (_NCOLB + _PK - 1) // _PK
_NROW = _PBLK * _PGRID
_PSH = _PBLK.bit_length() - 1


def _pack_body(u0_ref, u1_ref, u2_ref, u3_ref, o_ref):
    eye = (jax.lax.broadcasted_iota(jnp.int32, (128, 128), 0)
           == jax.lax.broadcasted_iota(jnp.int32, (128, 128), 1)).astype(jnp.float32)
    ubig = jnp.concatenate(
        [u0_ref[...], u1_ref[...], u2_ref[...], u3_ref[...]], axis=0)
    o_ref[...] = jax.lax.dot_general(
        ubig, eye, (((0,), (0,)), ((), ())),
        preferred_element_type=jnp.float32)


_pack = pl.pallas_call(
    _pack_body,
    grid=(_PGRID,),
    in_specs=[pl.BlockSpec(
        (_D, _PBLK),
        lambda j, k=k: (0, jnp.minimum(_PK * j + k, _NCOLB - 1)))
        for k in range(_PK)],
    out_specs=pl.BlockSpec((_PBLK, 128), lambda j: (j, 0)),
    out_shape=jax.ShapeDtypeStruct((_NROW, 128), jnp.float32),
    compiler_params=pltpu.CompilerParams(fuse_transposed_lhs_in_matmul=True),
)


@functools.cache
def _sc_gather_fn():
    mesh = plsc.VectorSubcoreMesh(
        core_axis_name="c", subcore_axis_name="s",
        num_cores=_NC, num_subcores=_NS,
    )

    @functools.partial(
        pl.kernel,
        out_type=(
            jax.ShapeDtypeStruct((_B, 128), jnp.float32),
            jax.ShapeDtypeStruct((_B, 128), jnp.float32),
        ),
        mesh=mesh,
        compiler_params=pltpu.CompilerParams(use_tc_tiling_on_sc=True),
        scratch_types=[
            pltpu.VMEM((_BPW,), jnp.int32),
            pltpu.VMEM((_BPW,), jnp.int32),
            pltpu.VMEM((_CH,), jnp.int32),
            pltpu.VMEM((_CH,), jnp.int32),
            pltpu.VMEM((_CH, 128), jnp.float32),
            pltpu.VMEM((_CH, 128), jnp.float32),
            pltpu.SemaphoreType.DMA,
            pltpu.SemaphoreType.DMA,
        ],
    )
    def sc_gather(U_hbm, V_hbm, u_hbm, i_hbm, eu_hbm, ev_hbm,
                  uidx, iidx, urow, irow, gu, gv, sem_u, sem_v):
        wid = lax.axis_index("s") * _NC + lax.axis_index("c")
        base = wid * _BPW
        pltpu.sync_copy(u_hbm.at[pl.ds(base, _BPW)], uidx)
        pltpu.sync_copy(i_hbm.at[pl.ds(base, _BPW)], iidx)
        for c in range(_NCH):
            for k in range(_CH // 16):
                s = pl.ds(k * 16, 16)
                uu = uidx[pl.ds(c * _CH + k * 16, 16)]
                ii = iidx[pl.ds(c * _CH + k * 16, 16)]
                urow[s] = ((jax.lax.shift_right_logical(uu, 2) & ~(_PBLK - 1))
                           | (uu & (_PBLK - 1)))
                irow[s] = ((jax.lax.shift_right_logical(ii, 2) & ~(_PBLK - 1))
                           | (ii & (_PBLK - 1)))
            cu = pltpu.async_copy(U_hbm.at[urow], gu, sem_u)
            cv = pltpu.async_copy(V_hbm.at[irow], gv, sem_v)
            cu.wait()
            pltpu.sync_copy(gu, eu_hbm.at[pl.ds(base + c * _CH, _CH)])
            cv.wait()
            pltpu.sync_copy(gv, ev_hbm.at[pl.ds(base + c * _CH, _CH)])

    return sc_gather


_BLK = 2048


def _mlp_body(u_ref, i_ref, eu_ref, ev_ref,
              w1a_ref, w1b_ref, b1_ref, w2_ref, b2_ref, o_ref):
    ku = (jnp.transpose(jnp.reshape(u_ref[...], (1, _BLK))) >> _PSH) & 3
    ki = (jnp.transpose(jnp.reshape(i_ref[...], (1, _BLK))) >> _PSH) & 3
    lane = jax.lax.broadcasted_iota(jnp.int32, (1, 128), 1) >> 5
    mu = (ku == lane).astype(jnp.float32)
    mi = (ki == lane).astype(jnp.float32)
    e = eu_ref[...] * mu
    f = ev_ref[...] * mi
    h = jnp.dot(e, w1a_ref[...], preferred_element_type=jnp.float32)
    h = h + jnp.dot(f, w1b_ref[...], preferred_element_type=jnp.float32)
    h = jnp.maximum(h + b1_ref[...], 0.0)
    z = jnp.sum(h * w2_ref[...], axis=1) + b2_ref[0, 0]
    o_ref[...] = 1.0 / (1.0 + jnp.exp(-z))


_mlp = pl.pallas_call(
    _mlp_body,
    grid=(_B // _BLK,),
    in_specs=[
        pl.BlockSpec((1, 1, _BLK), lambda j: (j, 0, 0)),
        pl.BlockSpec((1, 1, _BLK), lambda j: (j, 0, 0)),
        pl.BlockSpec((_BLK, 128), lambda j: (j, 0)),
        pl.BlockSpec((_BLK, 128), lambda j: (j, 0)),
        pl.BlockSpec((128, _H), lambda j: (0, 0)),
        pl.BlockSpec((128, _H), lambda j: (0, 0)),
        pl.BlockSpec((1, _H), lambda j: (0, 0)),
        pl.BlockSpec((1, _H), lambda j: (0, 0)),
        pl.BlockSpec((1, 1), lambda j: (0, 0)),
    ],
    out_specs=pl.BlockSpec((_BLK,), lambda j: (j,)),
    out_shape=jax.ShapeDtypeStruct((_B,), jnp.float32),
)


def kernel(u, i, U, V, W1, b1, W2, b2):
    u = u.astype(jnp.int32)
    i = i.astype(jnp.int32)
    UT, VT = U.T, V.T
    U128 = _pack(UT, UT, UT, UT)
    V128 = _pack(VT, VT, VT, VT)
    eu128, ev128 = _sc_gather_fn()(U128, V128, u, i)
    w1a = jnp.tile(W1[:, :_D].T, (_PK, 1))
    w1b = jnp.tile(W1[:, _D:].T, (_PK, 1))
    return _mlp(u.reshape(_B // _BLK, 1, _BLK), i.reshape(_B // _BLK, 1, _BLK),
                eu128, ev128, w1a, w1b,
                b1.reshape(1, _H), W2, b2.reshape(1, 1))


# speedup vs baseline: 5.2110x; 1.2120x over previous
import functools

import jax
import jax.numpy as jnp
from jax import lax
from jax.experimental import pallas as pl
from jax.experimental.pallas import tpu as pltpu
from jax.experimental.pallas import tpu_sc as plsc

_NV = 1000000
_B = 16384
_D = 32
_H = 64
_NC = 2
_NS = 16
_NW = _NC * _NS
_BPW = _B // _NW
_CH = 256
_NCH = _BPW // _CH
_PK = 8
_PBLK = 8192
_NCOLB = (_NV + _PBLK - 1) // _PBLK
_PGRID = (_NCOLB + _PK - 1) // _PK
_NROW = _PBLK * _PGRID
_PSH = _PBLK.bit_length() - 1


def _pack_body(u0_ref, u1_ref, u2_ref, u3_ref,
               u4_ref, u5_ref, u6_ref, u7_ref, o_ref):
    eye = (jax.lax.broadcasted_iota(jnp.int32, (128, 128), 0)
           == jax.lax.broadcasted_iota(jnp.int32, (128, 128), 1)).astype(jnp.bfloat16)
    dn = (((0,), (0,)), ((), ()))
    lo = jnp.concatenate(
        [u0_ref[...], u2_ref[...], u4_ref[...], u6_ref[...]],
        axis=0).astype(jnp.bfloat16)
    hi = jnp.concatenate(
        [u1_ref[...], u3_ref[...], u5_ref[...], u7_ref[...]],
        axis=0).astype(jnp.bfloat16)
    tlo = jax.lax.dot_general(lo, eye, dn, preferred_element_type=jnp.float32)
    thi = jax.lax.dot_general(hi, eye, dn, preferred_element_type=jnp.float32)
    blo = jax.lax.bitcast_convert_type(tlo, jnp.uint32)
    bhi = jax.lax.bitcast_convert_type(thi, jnp.uint32)
    packed = (blo >> 16) | (bhi & jnp.uint32(0xFFFF0000))
    o_ref[...] = jax.lax.bitcast_convert_type(packed, jnp.float32)


_pack = pl.pallas_call(
    _pack_body,
    grid=(_PGRID,),
    in_specs=[pl.BlockSpec(
        (_D, _PBLK),
        lambda j, k=k: (0, jnp.minimum(_PK * j + k, _NCOLB - 1)))
        for k in range(_PK)],
    out_specs=pl.BlockSpec((_PBLK, 128), lambda j: (j, 0)),
    out_shape=jax.ShapeDtypeStruct((_NROW, 128), jnp.float32),
    compiler_params=pltpu.CompilerParams(fuse_transposed_lhs_in_matmul=True),
)


@functools.cache
def _sc_gather_fn():
    mesh = plsc.VectorSubcoreMesh(
        core_axis_name="c", subcore_axis_name="s",
        num_cores=_NC, num_subcores=_NS,
    )

    @functools.partial(
        pl.kernel,
        out_type=(
            jax.ShapeDtypeStruct((_B, 128), jnp.float32),
            jax.ShapeDtypeStruct((_B, 128), jnp.float32),
        ),
        mesh=mesh,
        compiler_params=pltpu.CompilerParams(use_tc_tiling_on_sc=True),
        scratch_types=[
            pltpu.VMEM((_BPW,), jnp.int32),
            pltpu.VMEM((_BPW,), jnp.int32),
            pltpu.VMEM((_CH,), jnp.int32),
            pltpu.VMEM((_CH,), jnp.int32),
            pltpu.VMEM((_CH, 128), jnp.float32),
            pltpu.VMEM((_CH, 128), jnp.float32),
            pltpu.SemaphoreType.DMA,
            pltpu.SemaphoreType.DMA,
        ],
    )
    def sc_gather(U_hbm, V_hbm, u_hbm, i_hbm, eu_hbm, ev_hbm,
                  uidx, iidx, urow, irow, gu, gv, sem_u, sem_v):
        wid = lax.axis_index("s") * _NC + lax.axis_index("c")
        base = wid * _BPW
        pltpu.sync_copy(u_hbm.at[pl.ds(base, _BPW)], uidx)
        pltpu.sync_copy(i_hbm.at[pl.ds(base, _BPW)], iidx)
        for c in range(_NCH):
            for k in range(_CH // 16):
                s = pl.ds(k * 16, 16)
                uu = uidx[pl.ds(c * _CH + k * 16, 16)]
                ii = iidx[pl.ds(c * _CH + k * 16, 16)]
                urow[s] = ((jax.lax.shift_right_logical(uu, 3) & ~(_PBLK - 1))
                           | (uu & (_PBLK - 1)))
                irow[s] = ((jax.lax.shift_right_logical(ii, 3) & ~(_PBLK - 1))
                           | (ii & (_PBLK - 1)))
            cu = pltpu.async_copy(U_hbm.at[urow], gu, sem_u)
            cv = pltpu.async_copy(V_hbm.at[irow], gv, sem_v)
            cu.wait()
            pltpu.sync_copy(gu, eu_hbm.at[pl.ds(base + c * _CH, _CH)])
            cv.wait()
            pltpu.sync_copy(gv, ev_hbm.at[pl.ds(base + c * _CH, _CH)])

    return sc_gather


_BLK = 2048


def _mlp_body(u_ref, i_ref, eu_ref, ev_ref,
              w1a_ref, w1b_ref, b1_ref, w2_ref, b2_ref, o_ref):
    uu = jnp.transpose(jnp.reshape(u_ref[...], (1, _BLK)))
    ii = jnp.transpose(jnp.reshape(i_ref[...], (1, _BLK)))
    lane = jax.lax.broadcasted_iota(jnp.int32, (1, 128), 1) >> 5
    mu = (((uu >> (_PSH + 1)) & 3) == lane).astype(jnp.float32)
    mi = (((ii >> (_PSH + 1)) & 3) == lane).astype(jnp.float32)
    be = jax.lax.bitcast_convert_type(eu_ref[...], jnp.uint32)
    bf = jax.lax.bitcast_convert_type(ev_ref[...], jnp.uint32)
    pe = ((uu >> _PSH) & 1) == 1
    pf = ((ii >> _PSH) & 1) == 1
    e = jax.lax.bitcast_convert_type(
        jnp.where(pe, be & jnp.uint32(0xFFFF0000), be << 16), jnp.float32)
    f = jax.lax.bitcast_convert_type(
        jnp.where(pf, bf & jnp.uint32(0xFFFF0000), bf << 16), jnp.float32)
    e = e * mu
    f = f * mi
    h = jnp.dot(e, w1a_ref[...], preferred_element_type=jnp.float32)
    h = h + jnp.dot(f, w1b_ref[...], preferred_element_type=jnp.float32)
    h = jnp.maximum(h + b1_ref[...], 0.0)
    z = jnp.sum(h * w2_ref[...], axis=1) + b2_ref[0, 0]
    o_ref[...] = 1.0 / (1.0 + jnp.exp(-z))


_mlp = pl.pallas_call(
    _mlp_body,
    grid=(_B // _BLK,),
    in_specs=[
        pl.BlockSpec((1, 1, _BLK), lambda j: (j, 0, 0)),
        pl.BlockSpec((1, 1, _BLK), lambda j: (j, 0, 0)),
        pl.BlockSpec((_BLK, 128), lambda j: (j, 0)),
        pl.BlockSpec((_BLK, 128), lambda j: (j, 0)),
        pl.BlockSpec((128, _H), lambda j: (0, 0)),
        pl.BlockSpec((128, _H), lambda j: (0, 0)),
        pl.BlockSpec((1, _H), lambda j: (0, 0)),
        pl.BlockSpec((1, _H), lambda j: (0, 0)),
        pl.BlockSpec((1, 1), lambda j: (0, 0)),
    ],
    out_specs=pl.BlockSpec((_BLK,), lambda j: (j,)),
    out_shape=jax.ShapeDtypeStruct((_B,), jnp.float32),
)


def kernel(u, i, U, V, W1, b1, W2, b2):
    u = u.astype(jnp.int32)
    i = i.astype(jnp.int32)
    UT, VT = U.T, V.T
    U128 = _pack(*([UT] * _PK))
    V128 = _pack(*([VT] * _PK))
    eu128, ev128 = _sc_gather_fn()(U128, V128, u, i)
    w1a = jnp.tile(W1[:, :_D].T, (128 // _D, 1))
    w1b = jnp.tile(W1[:, _D:].T, (128 // _D, 1))
    return _mlp(u.reshape(_B // _BLK, 1, _BLK), i.reshape(_B // _BLK, 1, _BLK),
                eu128, ev128, w1a, w1b,
                b1.reshape(1, _H), W2, b2.reshape(1, 1))
